# baseline (speedup 1.0000x reference)
import functools

import jax
import jax.numpy as jnp
import numpy as np
from jax import lax
from jax.experimental import pallas as pl
from jax.experimental.pallas import tpu as pltpu
from jax.experimental.pallas import tpu_sc as plsc

N = 10000
D_IN = 1433
H1, C1 = 10, 8
F1 = H1 * C1
SRCW1 = 96
DSTW1 = 16
W2COLS = 16
NACC = 10112
ROWS_PER_TILE = NACC // 16
ZROWS = ROWS_PER_TILE // 4
E_LOOPED = 160000 + N
CHUNK = 128
NCORES = 2
NWORK = 16 * NCORES
CH_PER_W = 42
NCHUNKS = 32 * CH_PER_W
EPAD = NCHUNKS * CHUNK
BLK = 128
GRID = (N + BLK - 1) // BLK
F32 = jnp.float32


BLK_A = 2000


def _tca_body(x_hbm, ws_ref, wd_ref, ts_ref, td_ref, xbuf, sems):
    def cp(b, slot):
        return pltpu.make_async_copy(
            x_hbm.at[pl.ds(b * BLK_A, BLK_A), :], xbuf.at[slot], sems.at[slot]
        )

    cp(0, 0).start()
    cp(1, 1).start()
    ws = ws_ref[...].astype(jnp.bfloat16)
    wd = wd_ref[...].astype(jnp.bfloat16)
    for b in range(5):
        slot = b % 2
        cp(b, slot).wait()
        xb = xbuf[slot].astype(jnp.bfloat16)
        ts_ref[pl.ds(b * BLK_A, BLK_A), :] = jnp.dot(
            xb, ws, preferred_element_type=F32
        ).astype(jnp.bfloat16)
        td_ref[pl.ds(b * BLK_A, BLK_A), :] = jnp.dot(
            xb, wd, preferred_element_type=F32
        ).astype(jnp.bfloat16)
        if b + 2 < 5:
            cp(b + 2, slot).start()


def _tc_a(x, ws1, wd1):
    return pl.pallas_call(
        _tca_body,
        grid=(1,),
        in_specs=[
            pl.BlockSpec(memory_space=pltpu.MemorySpace.HBM),
            pl.BlockSpec((D_IN, SRCW1), lambda i: (0, 0)),
            pl.BlockSpec((D_IN, 2 * DSTW1), lambda i: (0, 0)),
        ],
        out_specs=[
            pl.BlockSpec((N, SRCW1), lambda i: (0, 0)),
            pl.BlockSpec((N, 2 * DSTW1), lambda i: (0, 0)),
        ],
        out_shape=[
            jax.ShapeDtypeStruct((N, SRCW1), jnp.bfloat16),
            jax.ShapeDtypeStruct((N, 2 * DSTW1), jnp.bfloat16),
        ],
        scratch_shapes=[
            pltpu.VMEM((2, BLK_A, D_IN), F32),
            pltpu.SemaphoreType.DMA((2,)),
        ],
    )(x, ws1, wd1)


_MESH = plsc.VectorSubcoreMesh(
    core_axis_name="c", subcore_axis_name="s", num_cores=NCORES
)

TOTCH = NCHUNKS + 6


def _zero_shared(zbuf, acc, sid, width):
    zero = jnp.zeros((16,), F32)

    def zrow(r, carry):
        for k in range(width // 16):
            zbuf[r, pl.ds(16 * k, 16)] = zero
        return carry

    lax.fori_loop(0, ZROWS, zrow, 0)
    for q in range(4):
        pltpu.sync_copy(zbuf, acc.at[pl.ds(sid * ROWS_PER_TILE + q * ZROWS, ZROWS), :])


def _make_sc_kernel(width, tdt, adw, compute_chunk):
    @functools.partial(
        pl.kernel,
        mesh=_MESH,
        out_type=jax.ShapeDtypeStruct((NCORES, NACC, width), F32),
        scratch_types=[
            pltpu.VMEM((2, CHUNK), jnp.int32),
            pltpu.VMEM((2, CHUNK), jnp.int32),
            pltpu.VMEM((CHUNK,), jnp.int32),
            pltpu.VMEM((CHUNK,), jnp.int32),
            pltpu.VMEM((CHUNK, width), tdt),
            pltpu.VMEM((CHUNK, width), tdt),
            pltpu.VMEM((CHUNK, adw), tdt),
            pltpu.VMEM((CHUNK, adw), tdt),
            pltpu.VMEM((CHUNK, width), F32),
            pltpu.VMEM((CHUNK, width), F32),
            pltpu.VMEM((ZROWS, width), F32),
            pltpu.VMEM_SHARED((NACC, width), F32),
        ] + [pltpu.SemaphoreType.DMA] * 8,
        compiler_params=pltpu.CompilerParams(use_tc_tiling_on_sc=False, needs_layout_passes=False),
    )
    def sc_kernel(sd_hbm, ts_hbm, td_hbm, out_hbm,
                  sd0, sd1, scd0, scd1, rows0, rows1, ad0, ad1, out0, out1,
                  zbuf, acc,
                  sdsem0, sdsem1, rg0, rg1, adsem0, adsem1, sc0, sc1):
        cid = lax.axis_index("c")
        sid = lax.axis_index("s")
        _zero_shared(zbuf, acc, sid, width)
        plsc.subcore_barrier()

        sds = (sd0, sd1)
        scds = (scd0, scd1)
        rowss = (rows0, rows1)
        ads = (ad0, ad1)
        outs = (out0, out1)
        sdsems = (sdsem0, sdsem1)
        rgs = (rg0, rg1)
        adsems = (adsem0, adsem1)
        scs = (sc0, sc1)
        cbase = 2 * sid * CH_PER_W + cid

        def iteration(g, s, skip_scwait):
            o = 1 - s
            pltpu.make_async_copy(sd_hbm.at[cbase + 2 * (g + 1)], sds[o], sdsems[o]).wait()
            pltpu.async_copy(ts_hbm.at[sds[o].at[0]], rowss[o], rgs[o])
            pltpu.async_copy(td_hbm.at[sds[o].at[1]], ads[o], adsems[o])
            pltpu.make_async_copy(ts_hbm.at[sds[s].at[0]], rowss[s], rgs[s]).wait()
            pltpu.make_async_copy(td_hbm.at[sds[s].at[1]], ads[s], adsems[s]).wait()
            if not skip_scwait:
                pltpu.make_async_copy(outs[s], acc.at[scds[s]], scs[s]).wait()
            for k in range(CHUNK // 16):
                scds[s][pl.ds(16 * k, 16)] = sds[s][1, pl.ds(16 * k, 16)]
            pltpu.async_copy(sd_hbm.at[cbase + 2 * (g + 2)], sds[s], sdsems[s])
            compute_chunk(rowss[s], ads[s], outs[s])
            pltpu.async_copy(outs[s], acc.at[scds[s]], scs[s], add=True)

        pltpu.async_copy(sd_hbm.at[cbase], sd0, sdsem0)
        pltpu.async_copy(sd_hbm.at[cbase + 2], sd1, sdsem1)
        pltpu.make_async_copy(sd_hbm.at[cbase], sd0, sdsem0).wait()
        pltpu.async_copy(ts_hbm.at[sd0.at[0]], rows0, rg0)
        pltpu.async_copy(td_hbm.at[sd0.at[1]], ad0, adsem0)
        iteration(0, 0, True)
        iteration(1, 1, True)

        def pair_body(g2, carry):
            iteration(2 * g2, 0, False)
            iteration(2 * g2 + 1, 1, False)
            return carry

        lax.fori_loop(1, CH_PER_W // 2, pair_body, 0)

        pltpu.make_async_copy(out0, acc.at[scd0], sc0).wait()
        pltpu.make_async_copy(out1, acc.at[scd1], sc1).wait()
        pltpu.make_async_copy(ts_hbm.at[sd0.at[0]], rows0, rg0).wait()
        pltpu.make_async_copy(td_hbm.at[sd0.at[1]], ad0, adsem0).wait()
        pltpu.make_async_copy(sd_hbm.at[cbase], sd1, sdsem1).wait()

        plsc.subcore_barrier()
        pltpu.sync_copy(
            acc.at[pl.ds(sid * ROWS_PER_TILE, ROWS_PER_TILE), :],
            out_hbm.at[cid, pl.ds(sid * ROWS_PER_TILE, ROWS_PER_TILE), :],
        )

    return sc_kernel


def _compute_chunk1(rows, ad, out):
    lanes = lax.iota(jnp.int32, 16)
    pats = [jnp.where(lanes < 8, 2 * v, 2 * v + 1) for v in range(5)]
    ilv = plsc.PackFormat.INTERLEAVED

    @plsc.parallel_loop(0, CHUNK, 1, unroll=8)
    def edge(e):
        adv, _ = plsc.unpack(ad[e, :], format=ilv)
        h0, h1 = plsc.unpack(rows[e, pl.ds(0, 32)], format=ilv)
        h2, h3 = plsc.unpack(rows[e, pl.ds(32, 32)], format=ilv)
        h4, asv = plsc.unpack(rows[e, pl.ds(64, 32)], format=ilv)
        s = asv + adv
        w = jnp.exp(jnp.where(s > 0, s, 0.2 * s))
        out[e, pl.ds(F1, 16)] = w
        for v, hv in enumerate((h0, h1, h2, h3, h4)):
            wv = jnp.take_along_axis(w, pats[v], axis=0)
            out[e, pl.ds(16 * v, 16)] = hv * wv


def _compute_chunk2(rows, ad, out):
    lane8 = jnp.full((16,), 8, dtype=jnp.int32)

    @plsc.parallel_loop(0, CHUNK, 1, unroll=8)
    def edge(e):
        sv = rows[e, :]
        s = sv + ad[e, :]
        w = jnp.exp(jnp.where(s > 0, s, 0.2 * s))
        out[e, :] = sv * jnp.take_along_axis(w, lane8, axis=0)


_sc_layer1 = _make_sc_kernel(SRCW1, jnp.bfloat16, 32, _compute_chunk1)
_sc_layer2 = _make_sc_kernel(W2COLS, F32, W2COLS, _compute_chunk2)


def _tcb_body(p_ref, psel_ref, ex_ref, b1_ref, wc2_ref, s_ref, d_ref, c_ref,
              ts2_ref, td2_ref):
    a = p_ref[0]
    for q in range(1, NCORES):
        a = a + p_ref[q]
    msg = a[:, :F1]
    den = jnp.dot(a, psel_ref[...], preferred_element_type=F32)
    rec = 1.0 / (den + 1e-16)
    recx = jnp.dot(rec, ex_ref[...], preferred_element_type=F32)
    h1 = msg * recx + b1_ref[...]
    h1a = jnp.where(h1 > 0, h1, jnp.exp(h1) - 1.0)
    t2 = jnp.dot(h1a, wc2_ref[...], preferred_element_type=F32)
    ts2_ref[...] = jnp.dot(t2, s_ref[...], preferred_element_type=F32) + c_ref[...]
    td2_ref[...] = jnp.dot(t2, d_ref[...], preferred_element_type=F32)


def _tc_b(p1, psel, ex, b1row, wc2, smat, dmat, cvec):
    return pl.pallas_call(
        _tcb_body,
        grid=(1,),
        in_specs=[
            pl.BlockSpec((NCORES, N, SRCW1), lambda i: (0, 0, 0)),
            pl.BlockSpec((SRCW1, H1), lambda i: (0, 0)),
            pl.BlockSpec((H1, F1), lambda i: (0, 0)),
            pl.BlockSpec((1, F1), lambda i: (0, 0)),
            pl.BlockSpec((F1, 9), lambda i: (0, 0)),
            pl.BlockSpec((9, W2COLS), lambda i: (0, 0)),
            pl.BlockSpec((9, W2COLS), lambda i: (0, 0)),
            pl.BlockSpec((1, W2COLS), lambda i: (0, 0)),
        ],
        out_specs=[
            pl.BlockSpec((N, W2COLS), lambda i: (0, 0)),
            pl.BlockSpec((N, W2COLS), lambda i: (0, 0)),
        ],
        out_shape=[
            jax.ShapeDtypeStruct((N, W2COLS), F32),
            jax.ShapeDtypeStruct((N, W2COLS), F32),
        ],
    )(p1, psel, ex, b1row, wc2, smat, dmat, cvec)


def _tcc_body(p_ref, b2_ref, o_ref):
    a = p_ref[0]
    for q in range(1, NCORES):
        a = a + p_ref[q]
    logits = a[:, :7] / (a[:, 7:8] + 1e-16) + b2_ref[...]
    m = jnp.max(logits, axis=-1, keepdims=True)
    z = logits - m
    lse = jnp.log(jnp.sum(jnp.exp(z), axis=-1, keepdims=True))
    o_ref[...] = z - lse


def _tc_c(p2, b2row):
    return pl.pallas_call(
        _tcc_body,
        grid=(1,),
        in_specs=[
            pl.BlockSpec((NCORES, N, W2COLS), lambda i: (0, 0, 0)),
            pl.BlockSpec((1, 7), lambda i: (0, 0)),
        ],
        out_specs=pl.BlockSpec((N, 7), lambda i: (0, 0)),
        out_shape=jax.ShapeDtypeStruct((N, 7), F32),
    )(p2, b2row)


def kernel(x, edge_index, W1, att_src1, att_dst1, b1, W2, att_src2, att_dst2, b2):
    eye10 = jnp.eye(H1, dtype=F32)
    As1 = (att_src1[:, :, None] * eye10[:, None, :]).reshape(F1, H1)
    Ad1 = (att_dst1[:, :, None] * eye10[:, None, :]).reshape(F1, H1)
    zpad = jnp.zeros((D_IN, 6), F32)
    ws1 = jnp.concatenate([W1, W1 @ As1, zpad], axis=1)
    perm = np.empty((SRCW1,), np.int32)
    for k in range(SRCW1 // 32):
        for j in range(16):
            perm[32 * k + 2 * j] = 32 * k + j
            perm[32 * k + 2 * j + 1] = 32 * k + 16 + j
    ws1 = ws1[:, perm]
    wd16 = jnp.concatenate([W1 @ Ad1, zpad], axis=1)
    wd1 = jnp.zeros((D_IN, 2 * DSTW1), F32).at[:, 0::2].set(wd16)

    wc2 = jnp.concatenate(
        [W2, W2 @ att_src2[0][:, None], W2 @ att_dst2[0][:, None]], axis=1
    )

    psel = jnp.asarray(
        np.concatenate([np.zeros((F1, H1)), np.eye(H1), np.zeros((6, H1))], axis=0),
        F32,
    )
    ex = jnp.asarray(np.repeat(np.eye(H1), C1, axis=1), F32)
    smat_np = np.zeros((9, W2COLS), np.float32)
    for j in range(7):
        smat_np[j, j] = 1.0
    smat_np[7, 8:] = 1.0
    smat = jnp.asarray(smat_np)
    dmat_np = np.zeros((9, W2COLS), np.float32)
    dmat_np[8, :] = 1.0
    dmat = jnp.asarray(dmat_np)
    cvec_np = np.zeros((1, W2COLS), np.float32)
    cvec_np[0, 7] = 1.0
    cvec = jnp.asarray(cvec_np)

    loops = jnp.arange(N, dtype=jnp.int32)
    npad = TOTCH * CHUNK - E_LOOPED
    src = jnp.concatenate(
        [edge_index[0].astype(jnp.int32), loops, jnp.zeros((npad,), jnp.int32)]
    )
    dst = jnp.concatenate(
        [edge_index[1].astype(jnp.int32), loops, jnp.full((npad,), N, jnp.int32)]
    )
    sd = jnp.stack(
        [src.reshape(TOTCH, CHUNK), dst.reshape(TOTCH, CHUNK)], axis=1
    )

    tsrc1, tdst1 = _tc_a(x, ws1, wd1)
    p1 = _sc_layer1(sd, tsrc1, tdst1)
    tsrc2, tdst2 = _tc_b(p1, psel, ex, b1.reshape(1, F1), wc2, smat, dmat, cvec)
    p2 = _sc_layer2(sd, tsrc2, tdst2)
    return _tc_c(p2, b2.reshape(1, 7))


# speedup vs baseline: 1.0005x; 1.0005x over previous
import functools

import jax
import jax.numpy as jnp
import numpy as np
from jax import lax
from jax.experimental import pallas as pl
from jax.experimental.pallas import tpu as pltpu
from jax.experimental.pallas import tpu_sc as plsc

N = 10000
D_IN = 1433
H1, C1 = 10, 8
F1 = H1 * C1
SRCW1 = 96
DSTW1 = 16
W2COLS = 16
NACC = 10112
ROWS_PER_TILE = NACC // 16
ZROWS = ROWS_PER_TILE // 4
E_LOOPED = 160000 + N
CHUNK = 128
NCORES = 2
NWORK = 16 * NCORES
CH_PER_W = 42
NCHUNKS = 32 * CH_PER_W
EPAD = NCHUNKS * CHUNK
BLK = 128
GRID = (N + BLK - 1) // BLK
F32 = jnp.float32


BLK_A = 2000


def _tca_body(x_hbm, ws_ref, wd_ref, ts_ref, td_ref, xbuf, sems):
    def cp(b, slot):
        return pltpu.make_async_copy(
            x_hbm.at[pl.ds(b * BLK_A, BLK_A), :], xbuf.at[slot], sems.at[slot]
        )

    cp(0, 0).start()
    cp(1, 1).start()
    ws = ws_ref[...].astype(jnp.bfloat16)
    wd = wd_ref[...].astype(jnp.bfloat16)
    for b in range(5):
        slot = b % 2
        cp(b, slot).wait()
        xb = xbuf[slot].astype(jnp.bfloat16)
        ts_ref[pl.ds(b * BLK_A, BLK_A), :] = jnp.dot(
            xb, ws, preferred_element_type=F32
        ).astype(jnp.bfloat16)
        td_ref[pl.ds(b * BLK_A, BLK_A), :] = jnp.dot(
            xb, wd, preferred_element_type=F32
        ).astype(jnp.bfloat16)
        if b + 2 < 5:
            cp(b + 2, slot).start()


def _tc_a(x, ws1, wd1):
    return pl.pallas_call(
        _tca_body,
        grid=(1,),
        in_specs=[
            pl.BlockSpec(memory_space=pltpu.MemorySpace.HBM),
            pl.BlockSpec((D_IN, SRCW1), lambda i: (0, 0)),
            pl.BlockSpec((D_IN, 2 * DSTW1), lambda i: (0, 0)),
        ],
        out_specs=[
            pl.BlockSpec((N, SRCW1), lambda i: (0, 0)),
            pl.BlockSpec((N, 2 * DSTW1), lambda i: (0, 0)),
        ],
        out_shape=[
            jax.ShapeDtypeStruct((N, SRCW1), jnp.bfloat16),
            jax.ShapeDtypeStruct((N, 2 * DSTW1), jnp.bfloat16),
        ],
        scratch_shapes=[
            pltpu.VMEM((2, BLK_A, D_IN), F32),
            pltpu.SemaphoreType.DMA((2,)),
        ],
    )(x, ws1, wd1)


_MESH = plsc.VectorSubcoreMesh(
    core_axis_name="c", subcore_axis_name="s", num_cores=NCORES
)

TOTCH = NCHUNKS + 6


def _zero_shared(zbuf, acc, sid, width):
    zero = jnp.zeros((16,), F32)

    def zrow(r, carry):
        for k in range(width // 16):
            zbuf[r, pl.ds(16 * k, 16)] = zero
        return carry

    lax.fori_loop(0, ZROWS, zrow, 0)
    for q in range(4):
        pltpu.sync_copy(zbuf, acc.at[pl.ds(sid * ROWS_PER_TILE + q * ZROWS, ZROWS), :])


def _make_sc_kernel(width, tdt, adw, compute_chunk):
    @functools.partial(
        pl.kernel,
        mesh=_MESH,
        out_type=jax.ShapeDtypeStruct((NCORES, NACC, width), F32),
        scratch_types=[
            pltpu.VMEM((2, CHUNK), jnp.int32),
            pltpu.VMEM((2, CHUNK), jnp.int32),
            pltpu.VMEM((CHUNK,), jnp.int32),
            pltpu.VMEM((CHUNK,), jnp.int32),
            pltpu.VMEM((CHUNK, width), tdt),
            pltpu.VMEM((CHUNK, width), tdt),
            pltpu.VMEM((CHUNK, adw), tdt),
            pltpu.VMEM((CHUNK, adw), tdt),
            pltpu.VMEM((CHUNK, width), F32),
            pltpu.VMEM((CHUNK, width), F32),
            pltpu.VMEM((ZROWS, width), F32),
            pltpu.VMEM_SHARED((NACC, width), F32),
        ] + [pltpu.SemaphoreType.DMA] * 8,
        compiler_params=pltpu.CompilerParams(use_tc_tiling_on_sc=False, needs_layout_passes=False),
    )
    def sc_kernel(sd_hbm, ts_hbm, td_hbm, out_hbm,
                  sd0, sd1, scd0, scd1, rows0, rows1, ad0, ad1, out0, out1,
                  zbuf, acc,
                  sdsem0, sdsem1, rg0, rg1, adsem0, adsem1, sc0, sc1):
        cid = lax.axis_index("c")
        sid = lax.axis_index("s")
        _zero_shared(zbuf, acc, sid, width)
        plsc.subcore_barrier()

        sds = (sd0, sd1)
        scds = (scd0, scd1)
        rowss = (rows0, rows1)
        ads = (ad0, ad1)
        outs = (out0, out1)
        sdsems = (sdsem0, sdsem1)
        rgs = (rg0, rg1)
        adsems = (adsem0, adsem1)
        scs = (sc0, sc1)
        cbase = 2 * sid * CH_PER_W + cid

        def iteration(g, s, skip_scwait):
            o = 1 - s
            pltpu.make_async_copy(sd_hbm.at[cbase + 2 * (g + 1)], sds[o], sdsems[o]).wait()
            pltpu.async_copy(ts_hbm.at[sds[o].at[0]], rowss[o], rgs[o])
            pltpu.async_copy(td_hbm.at[sds[o].at[1]], ads[o], adsems[o])
            pltpu.make_async_copy(ts_hbm.at[sds[s].at[0]], rowss[s], rgs[s]).wait()
            pltpu.make_async_copy(td_hbm.at[sds[s].at[1]], ads[s], adsems[s]).wait()
            if not skip_scwait:
                pltpu.make_async_copy(outs[s], acc.at[scds[s]], scs[s]).wait()
            for k in range(CHUNK // 16):
                scds[s][pl.ds(16 * k, 16)] = sds[s][1, pl.ds(16 * k, 16)]
            pltpu.async_copy(sd_hbm.at[cbase + 2 * (g + 2)], sds[s], sdsems[s])
            compute_chunk(rowss[s], ads[s], outs[s])
            pltpu.async_copy(outs[s], acc.at[scds[s]], scs[s], add=True)

        pltpu.async_copy(sd_hbm.at[cbase], sd0, sdsem0)
        pltpu.async_copy(sd_hbm.at[cbase + 2], sd1, sdsem1)
        pltpu.make_async_copy(sd_hbm.at[cbase], sd0, sdsem0).wait()
        pltpu.async_copy(ts_hbm.at[sd0.at[0]], rows0, rg0)
        pltpu.async_copy(td_hbm.at[sd0.at[1]], ad0, adsem0)
        iteration(0, 0, True)
        iteration(1, 1, True)

        def pair_body(g2, carry):
            iteration(2 * g2, 0, False)
            iteration(2 * g2 + 1, 1, False)
            return carry

        lax.fori_loop(1, CH_PER_W // 2, pair_body, 0)

        pltpu.make_async_copy(out0, acc.at[scd0], sc0).wait()
        pltpu.make_async_copy(out1, acc.at[scd1], sc1).wait()
        pltpu.make_async_copy(ts_hbm.at[sd0.at[0]], rows0, rg0).wait()
        pltpu.make_async_copy(td_hbm.at[sd0.at[1]], ad0, adsem0).wait()
        pltpu.make_async_copy(sd_hbm.at[cbase], sd1, sdsem1).wait()

        plsc.subcore_barrier()
        pltpu.sync_copy(
            acc.at[pl.ds(sid * ROWS_PER_TILE, ROWS_PER_TILE), :],
            out_hbm.at[cid, pl.ds(sid * ROWS_PER_TILE, ROWS_PER_TILE), :],
        )

    return sc_kernel


def _compute_chunk1(rows, ad, out):
    lanes = lax.iota(jnp.int32, 16)
    pats = [jnp.where(lanes < 8, 2 * v, 2 * v + 1) for v in range(5)]
    ilv = plsc.PackFormat.INTERLEAVED

    @plsc.parallel_loop(0, CHUNK, 1, unroll=4)
    def edge(e):
        adv, _ = plsc.unpack(ad[e, :], format=ilv)
        h0, h1 = plsc.unpack(rows[e, pl.ds(0, 32)], format=ilv)
        h2, h3 = plsc.unpack(rows[e, pl.ds(32, 32)], format=ilv)
        h4, asv = plsc.unpack(rows[e, pl.ds(64, 32)], format=ilv)
        s = asv + adv
        w = jnp.exp(jnp.where(s > 0, s, 0.2 * s))
        out[e, pl.ds(F1, 16)] = w
        for v, hv in enumerate((h0, h1, h2, h3, h4)):
            wv = jnp.take_along_axis(w, pats[v], axis=0)
            out[e, pl.ds(16 * v, 16)] = hv * wv


def _compute_chunk2(rows, ad, out):
    lane8 = jnp.full((16,), 8, dtype=jnp.int32)

    @plsc.parallel_loop(0, CHUNK, 1, unroll=4)
    def edge(e):
        sv = rows[e, :]
        s = sv + ad[e, :]
        w = jnp.exp(jnp.where(s > 0, s, 0.2 * s))
        out[e, :] = sv * jnp.take_along_axis(w, lane8, axis=0)


_sc_layer1 = _make_sc_kernel(SRCW1, jnp.bfloat16, 32, _compute_chunk1)
_sc_layer2 = _make_sc_kernel(W2COLS, F32, W2COLS, _compute_chunk2)


def _tcb_body(p_ref, psel_ref, ex_ref, b1_ref, wc2_ref, s_ref, d_ref, c_ref,
              ts2_ref, td2_ref):
    a = p_ref[0]
    for q in range(1, NCORES):
        a = a + p_ref[q]
    msg = a[:, :F1]
    den = jnp.dot(a, psel_ref[...], preferred_element_type=F32)
    rec = 1.0 / (den + 1e-16)
    recx = jnp.dot(rec, ex_ref[...], preferred_element_type=F32)
    h1 = msg * recx + b1_ref[...]
    h1a = jnp.where(h1 > 0, h1, jnp.exp(h1) - 1.0)
    t2 = jnp.dot(h1a, wc2_ref[...], preferred_element_type=F32)
    ts2_ref[...] = jnp.dot(t2, s_ref[...], preferred_element_type=F32) + c_ref[...]
    td2_ref[...] = jnp.dot(t2, d_ref[...], preferred_element_type=F32)


def _tc_b(p1, psel, ex, b1row, wc2, smat, dmat, cvec):
    return pl.pallas_call(
        _tcb_body,
        grid=(1,),
        in_specs=[
            pl.BlockSpec((NCORES, N, SRCW1), lambda i: (0, 0, 0)),
            pl.BlockSpec((SRCW1, H1), lambda i: (0, 0)),
            pl.BlockSpec((H1, F1), lambda i: (0, 0)),
            pl.BlockSpec((1, F1), lambda i: (0, 0)),
            pl.BlockSpec((F1, 9), lambda i: (0, 0)),
            pl.BlockSpec((9, W2COLS), lambda i: (0, 0)),
            pl.BlockSpec((9, W2COLS), lambda i: (0, 0)),
            pl.BlockSpec((1, W2COLS), lambda i: (0, 0)),
        ],
        out_specs=[
            pl.BlockSpec((N, W2COLS), lambda i: (0, 0)),
            pl.BlockSpec((N, W2COLS), lambda i: (0, 0)),
        ],
        out_shape=[
            jax.ShapeDtypeStruct((N, W2COLS), F32),
            jax.ShapeDtypeStruct((N, W2COLS), F32),
        ],
    )(p1, psel, ex, b1row, wc2, smat, dmat, cvec)


def _tcc_body(p_ref, b2_ref, o_ref):
    a = p_ref[0]
    for q in range(1, NCORES):
        a = a + p_ref[q]
    logits = a[:, :7] / (a[:, 7:8] + 1e-16) + b2_ref[...]
    m = jnp.max(logits, axis=-1, keepdims=True)
    z = logits - m
    lse = jnp.log(jnp.sum(jnp.exp(z), axis=-1, keepdims=True))
    o_ref[...] = z - lse


def _tc_c(p2, b2row):
    return pl.pallas_call(
        _tcc_body,
        grid=(1,),
        in_specs=[
            pl.BlockSpec((NCORES, N, W2COLS), lambda i: (0, 0, 0)),
            pl.BlockSpec((1, 7), lambda i: (0, 0)),
        ],
        out_specs=pl.BlockSpec((N, 7), lambda i: (0, 0)),
        out_shape=jax.ShapeDtypeStruct((N, 7), F32),
    )(p2, b2row)


def kernel(x, edge_index, W1, att_src1, att_dst1, b1, W2, att_src2, att_dst2, b2):
    eye10 = jnp.eye(H1, dtype=F32)
    As1 = (att_src1[:, :, None] * eye10[:, None, :]).reshape(F1, H1)
    Ad1 = (att_dst1[:, :, None] * eye10[:, None, :]).reshape(F1, H1)
    zpad = jnp.zeros((D_IN, 6), F32)
    ws1 = jnp.concatenate([W1, W1 @ As1, zpad], axis=1)
    perm = np.empty((SRCW1,), np.int32)
    for k in range(SRCW1 // 32):
        for j in range(16):
            perm[32 * k + 2 * j] = 32 * k + j
            perm[32 * k + 2 * j + 1] = 32 * k + 16 + j
    ws1 = ws1[:, perm]
    wd16 = jnp.concatenate([W1 @ Ad1, zpad], axis=1)
    wd1 = jnp.zeros((D_IN, 2 * DSTW1), F32).at[:, 0::2].set(wd16)

    wc2 = jnp.concatenate(
        [W2, W2 @ att_src2[0][:, None], W2 @ att_dst2[0][:, None]], axis=1
    )

    psel = jnp.asarray(
        np.concatenate([np.zeros((F1, H1)), np.eye(H1), np.zeros((6, H1))], axis=0),
        F32,
    )
    ex = jnp.asarray(np.repeat(np.eye(H1), C1, axis=1), F32)
    smat_np = np.zeros((9, W2COLS), np.float32)
    for j in range(7):
        smat_np[j, j] = 1.0
    smat_np[7, 8:] = 1.0
    smat = jnp.asarray(smat_np)
    dmat_np = np.zeros((9, W2COLS), np.float32)
    dmat_np[8, :] = 1.0
    dmat = jnp.asarray(dmat_np)
    cvec_np = np.zeros((1, W2COLS), np.float32)
    cvec_np[0, 7] = 1.0
    cvec = jnp.asarray(cvec_np)

    loops = jnp.arange(N, dtype=jnp.int32)
    npad = TOTCH * CHUNK - E_LOOPED
    src = jnp.concatenate(
        [edge_index[0].astype(jnp.int32), loops, jnp.zeros((npad,), jnp.int32)]
    )
    dst = jnp.concatenate(
        [edge_index[1].astype(jnp.int32), loops, jnp.full((npad,), N, jnp.int32)]
    )
    sd = jnp.stack(
        [src.reshape(TOTCH, CHUNK), dst.reshape(TOTCH, CHUNK)], axis=1
    )

    tsrc1, tdst1 = _tc_a(x, ws1, wd1)
    p1 = _sc_layer1(sd, tsrc1, tdst1)
    tsrc2, tdst2 = _tc_b(p1, psel, ex, b1.reshape(1, F1), wc2, smat, dmat, cvec)
    p2 = _sc_layer2(sd, tsrc2, tdst2)
    return _tc_c(p2, b2.reshape(1, 7))
